# baseline (speedup 1.0000x reference)
import jax
import jax.numpy as jnp
from jax import lax
from jax.experimental import pallas as pl
from jax.experimental.pallas import tpu as pltpu


def _linear_kernel(x_ref, w_ref, b_ref, o_ref):
    m = x_ref.shape[0]
    bm = m // 8
    for mo in range(0, m, m // 8):
        acc = lax.dot_general(
            x_ref[pl.ds(mo, bm), :], w_ref[...],
            dimension_numbers=(((1,), (1,)), ((), ())),
            preferred_element_type=jnp.float32,
            precision=lax.Precision.DEFAULT,
        )
        o_ref[pl.ds(mo, bm), :] = acc + b_ref[...]


def kernel(x, W, b):
    M, K = x.shape
    N = W.shape[0]
    BN = 256
    b2 = b.reshape(1, N)
    out = pl.pallas_call(
        _linear_kernel,
        grid=(N // BN,),
        in_specs=[
            pl.BlockSpec((M, K), lambda i: (0, 0)),
            pl.BlockSpec((BN, K), lambda i: (i, 0)),
            pl.BlockSpec((1, BN), lambda i: (0, i)),
        ],
        out_specs=pl.BlockSpec((M, BN), lambda i: (0, i)),
        out_shape=jax.ShapeDtypeStruct((M, N), jnp.float32),
        compiler_params=pltpu.CompilerParams(
            dimension_semantics=("arbitrary",),
        ),
    )(x, W, b2)
    return out


# speedup vs baseline: 1.0064x; 1.0064x over previous
import jax
import jax.numpy as jnp
from jax import lax
from jax.experimental import pallas as pl
from jax.experimental.pallas import tpu as pltpu


def _linear_kernel(x_ref, w_ref, b_ref, o_ref):
    m = x_ref.shape[0]
    bm = max(m // 4, 8)
    for mo in range(0, m, bm):
        acc = lax.dot_general(
            x_ref[pl.ds(mo, bm), :], w_ref[...],
            dimension_numbers=(((1,), (1,)), ((), ())),
            preferred_element_type=jnp.float32,
            precision=lax.Precision.DEFAULT,
        )
        o_ref[pl.ds(mo, bm), :] = acc + b_ref[...]


def _pick_bn(n):
    for bn in (256, 128):
        if n % bn == 0:
            return bn
    return n


def kernel(x, W, b):
    M, K = x.shape
    N = W.shape[0]
    BN = _pick_bn(N)
    b2 = b.reshape(1, N)
    out = pl.pallas_call(
        _linear_kernel,
        grid=(N // BN,),
        in_specs=[
            pl.BlockSpec((M, K), lambda i: (0, 0)),
            pl.BlockSpec((BN, K), lambda i: (i, 0)),
            pl.BlockSpec((1, BN), lambda i: (0, i)),
        ],
        out_specs=pl.BlockSpec((M, BN), lambda i: (0, i)),
        out_shape=jax.ShapeDtypeStruct((M, N), jnp.float32),
        compiler_params=pltpu.CompilerParams(
            dimension_semantics=("arbitrary",),
        ),
    )(x, W, b2)
    return out
